# Initial kernel scaffold; baseline (speedup 1.0000x reference)
import jax
import jax.numpy as jnp
from jax.experimental import pallas as pl


def kernel(x, edge_index, W1, b1, W2, b2):
    raise NotImplementedError("write your pallas kernel here")


# speedup vs baseline: 12.5880x; 12.5880x over previous
import functools

import jax
import jax.numpy as jnp
from jax import lax
from jax.experimental import pallas as pl
from jax.experimental.pallas import tpu as pltpu
from jax.experimental.pallas import tpu_sc as plsc

N_NODES = 10000
N_PAD = 10240
N_EDGES = 320000
D_FEAT = 128
D_HID = 128
N_CLASSES = 64

NC = 2
NS = 16
NW = NC * NS
EPW = N_EDGES // NW
K = 80
CHUNKS = EPW // K
RPT = N_PAD // NS


def _sc_mesh():
    return plsc.VectorSubcoreMesh(core_axis_name="c", subcore_axis_name="s")


def _deg_body(col_hbm, zeros_hbm, out_hbm, cidx, ones, acc, sem):
    cid = lax.axis_index("c")
    sid = lax.axis_index("s")
    wid = cid * NS + sid
    pltpu.sync_copy(zeros_hbm, acc.at[pl.ds(sid * RPT, RPT)])
    for j in range(K // 16):
        ones[pl.ds(j * 16, 16)] = jnp.ones((16,), jnp.float32)
    plsc.subcore_barrier()

    def body(i, carry):
        chunk = wid * CHUNKS + i
        pltpu.sync_copy(col_hbm.at[chunk], cidx)
        pltpu.sync_copy(ones, acc.at[cidx], add=True)
        return carry

    lax.fori_loop(0, CHUNKS, body, 0)
    plsc.subcore_barrier()
    pltpu.sync_copy(acc.at[pl.ds(sid * RPT, RPT)],
                    out_hbm.at[cid, pl.ds(sid * RPT, RPT)])


def _make_deg_kernel():
    return pl.kernel(
        _deg_body,
        out_type=jax.ShapeDtypeStruct((NC, N_PAD), jnp.float32),
        mesh=_sc_mesh(),
        scratch_types=[
            pltpu.VMEM((K,), jnp.int32),
            pltpu.VMEM((K,), jnp.float32),
            pltpu.VMEM_SHARED((N_PAD,), jnp.float32),
            pltpu.SemaphoreType.DMA,
        ],
    )


def _prop_body(h_hbm, row_hbm, col_hbm, zeros_hbm, out_hbm,
               ridx, cidx, rows, acc, sem):
    cid = lax.axis_index("c")
    sid = lax.axis_index("s")
    wid = cid * NS + sid
    pltpu.sync_copy(zeros_hbm, acc.at[pl.ds(sid * RPT, RPT)])
    plsc.subcore_barrier()

    def body(i, carry):
        chunk = wid * CHUNKS + i
        pltpu.sync_copy(row_hbm.at[chunk], ridx)
        pltpu.async_copy(h_hbm.at[ridx], rows, sem).wait()
        pltpu.sync_copy(col_hbm.at[chunk], cidx)
        pltpu.sync_copy(rows, acc.at[cidx], add=True)
        return carry

    lax.fori_loop(0, CHUNKS, body, 0)
    plsc.subcore_barrier()
    pltpu.sync_copy(acc.at[pl.ds(sid * RPT, RPT)],
                    out_hbm.at[cid, pl.ds(sid * RPT, RPT)])


def _make_prop_kernel(d):
    return pl.kernel(
        _prop_body,
        out_type=jax.ShapeDtypeStruct((NC, N_PAD, d), jnp.float32),
        mesh=_sc_mesh(),
        scratch_types=[
            pltpu.VMEM((K,), jnp.int32),
            pltpu.VMEM((K,), jnp.int32),
            pltpu.VMEM((K, d), jnp.float32),
            pltpu.VMEM_SHARED((N_PAD, d), jnp.float32),
            pltpu.SemaphoreType.DMA,
        ],
    )


def _dis_body(p_ref, o_ref):
    deg = p_ref[0] + p_ref[1] + 1.0
    o_ref[...] = lax.rsqrt(deg)


def _mm1_body(x_ref, d_ref, w_ref, o_ref):
    xs = x_ref[...] * d_ref[...]
    o_ref[...] = jnp.dot(xs, w_ref[...], preferred_element_type=jnp.float32)


def _epi1_body(p0_ref, p1_ref, h_ref, d_ref, b_ref, w_ref, o_ref):
    s = p0_ref[...] + p1_ref[...] + h_ref[...]
    d = d_ref[...]
    h1 = jnp.maximum(d * s + b_ref[...], 0.0)
    o_ref[...] = jnp.dot(d * h1, w_ref[...],
                         preferred_element_type=jnp.float32)


def _epi2_body(p0_ref, p1_ref, h_ref, d_ref, b_ref, o_ref):
    s = p0_ref[:, :N_CLASSES] + p1_ref[:, :N_CLASSES] + h_ref[:, :N_CLASSES]
    z = d_ref[...] * s + b_ref[...]
    m = jnp.max(z, axis=1, keepdims=True)
    e = jnp.exp(z - m)
    lse = jnp.log(jnp.sum(e, axis=1, keepdims=True))
    o_ref[...] = z - m - lse


_R = 1024
_G = N_PAD // _R


def _row_specs(d):
    return pl.BlockSpec((_R, d), lambda i: (i, 0))


def _tc_mm1(x, dis1, w):
    return pl.pallas_call(
        _mm1_body,
        grid=(_G,),
        in_specs=[_row_specs(D_FEAT), pl.BlockSpec((_R, 1), lambda i: (i, 0)),
                  pl.BlockSpec((D_FEAT, D_HID), lambda i: (0, 0))],
        out_specs=_row_specs(D_HID),
        out_shape=jax.ShapeDtypeStruct((N_PAD, D_HID), jnp.float32),
    )(x, dis1, w)


def _tc_epi1(p0, p1, h, dis1, b, w):
    return pl.pallas_call(
        _epi1_body,
        grid=(_G,),
        in_specs=[_row_specs(D_HID), _row_specs(D_HID), _row_specs(D_HID),
                  pl.BlockSpec((_R, 1), lambda i: (i, 0)),
                  pl.BlockSpec((1, D_HID), lambda i: (0, 0)),
                  pl.BlockSpec((D_HID, D_HID), lambda i: (0, 0))],
        out_specs=_row_specs(D_HID),
        out_shape=jax.ShapeDtypeStruct((N_PAD, D_HID), jnp.float32),
    )(p0, p1, h, dis1, b, w)


def _tc_epi2(p0, p1, h, dis1, b):
    return pl.pallas_call(
        _epi2_body,
        grid=(_G,),
        in_specs=[_row_specs(D_HID), _row_specs(D_HID),
                  _row_specs(D_HID),
                  pl.BlockSpec((_R, 1), lambda i: (i, 0)),
                  pl.BlockSpec((1, N_CLASSES), lambda i: (0, 0))],
        out_specs=_row_specs(N_CLASSES),
        out_shape=jax.ShapeDtypeStruct((N_PAD, N_CLASSES), jnp.float32),
    )(p0, p1, h, dis1, b)


def _tc_dis(parts):
    return pl.pallas_call(
        _dis_body,
        out_shape=jax.ShapeDtypeStruct((N_PAD // 128, 128), jnp.float32),
    )(parts)


def kernel(x, edge_index, W1, b1, W2, b2):
    assert x.shape == (N_NODES, D_FEAT)
    assert edge_index.shape == (2, N_EDGES)

    row2d = edge_index[0].reshape(NW * CHUNKS, K)
    col2d = edge_index[1].reshape(NW * CHUNKS, K)
    x_pad = jnp.pad(x, ((0, N_PAD - N_NODES), (0, 0)))
    zeros1 = jnp.zeros((RPT,), jnp.float32)
    zeros128 = jnp.zeros((RPT, D_HID), jnp.float32)
    W2p = jnp.pad(W2, ((0, 0), (0, D_HID - N_CLASSES)))

    deg_parts = _make_deg_kernel()(col2d, zeros1)
    dis1 = _tc_dis(deg_parts.reshape(NC, N_PAD // 128, 128)).reshape(N_PAD, 1)

    h1p = _tc_mm1(x_pad, dis1, W1)
    parts1 = _make_prop_kernel(D_HID)(h1p, row2d, col2d, zeros128)
    h2p = _tc_epi1(parts1[0], parts1[1], h1p, dis1,
                   b1.reshape(1, D_HID), W2p)
    parts2 = _make_prop_kernel(D_HID)(h2p, row2d, col2d, zeros128)
    out = _tc_epi2(parts2[0], parts2[1], h2p, dis1,
                   b2.reshape(1, N_CLASSES))
    return out[:N_NODES]


# speedup vs baseline: 26.7138x; 2.1222x over previous
import jax
import jax.numpy as jnp
from jax import lax
from jax.experimental import pallas as pl
from jax.experimental.pallas import tpu as pltpu
from jax.experimental.pallas import tpu_sc as plsc

N_NODES = 10000
N_EDGES = 320000
D_FEAT = 128
D_HID = 128
N_CLASSES = 64

NC = 2
NS = 16
NW = NC * NS
EPW = N_EDGES // NW

N_PAD = 10112
RPT = N_PAD // NS

K = 50
CHUNKS = EPW // K
NBUF = 4

DEG_PAD = 10240
DRPT = DEG_PAD // NS
DK = 80
DCH = EPW // DK


def _sc_mesh():
    return plsc.VectorSubcoreMesh(core_axis_name="c", subcore_axis_name="s")


def _deg_body(col_hbm, zeros_hbm, out_hbm, cidx2, ones, acc, isem, ssem):
    cid = lax.axis_index("c")
    sid = lax.axis_index("s")
    wid = cid * NS + sid
    idx_cp = pltpu.async_copy(col_hbm.at[wid], cidx2, isem)
    pltpu.sync_copy(zeros_hbm, acc.at[pl.ds(sid * DRPT, DRPT)])
    for j in range(DK // 16):
        ones[pl.ds(j * 16, 16)] = jnp.ones((16,), jnp.float32)
    idx_cp.wait()
    plsc.subcore_barrier()

    def fire(i, carry):
        pltpu.async_copy(ones, acc.at[cidx2.at[i]], ssem, add=True)
        return carry

    def drain(i, carry):
        pltpu.make_async_copy(ones, acc.at[cidx2.at[i]], ssem).wait()
        return carry

    lax.fori_loop(0, DCH, fire, 0)
    lax.fori_loop(0, DCH, drain, 0)
    plsc.subcore_barrier()
    pltpu.sync_copy(acc.at[pl.ds(sid * DRPT, DRPT)],
                    out_hbm.at[cid, pl.ds(sid * DRPT, DRPT)])


def _make_deg_kernel():
    return pl.kernel(
        _deg_body,
        out_type=jax.ShapeDtypeStruct((NC, DEG_PAD), jnp.float32),
        mesh=_sc_mesh(),
        scratch_types=[
            pltpu.VMEM((DCH, DK), jnp.int32),
            pltpu.VMEM((DK,), jnp.float32),
            pltpu.VMEM_SHARED((DEG_PAD,), jnp.float32),
            pltpu.SemaphoreType.DMA,
            pltpu.SemaphoreType.DMA,
        ],
    )


def _prop_body(h_hbm, row_hbm, col_hbm, zeros_hbm, out_hbm,
               ridx, cidx, rows0, rows1, rows2, rows3, acc, zsem,
               rsem0, rsem1, rsem2, rsem3, csem0, csem1, csem2, csem3,
               gsem0, gsem1, gsem2, gsem3, ssem0, ssem1, ssem2, ssem3):
    cid = lax.axis_index("c")
    sid = lax.axis_index("s")
    wid = cid * NS + sid
    base = wid * CHUNKS
    rows = (rows0, rows1, rows2, rows3)
    rsem = (rsem0, rsem1, rsem2, rsem3)
    csem = (csem0, csem1, csem2, csem3)
    gsem = (gsem0, gsem1, gsem2, gsem3)
    ssem = (ssem0, ssem1, ssem2, ssem3)

    def load_ridx(i, b):
        pltpu.async_copy(row_hbm.at[base + i], ridx.at[b], rsem[b])

    def load_cidx(i, b):
        pltpu.async_copy(col_hbm.at[base + i], cidx.at[b], csem[b])

    def gather(b):
        pltpu.async_copy(h_hbm.at[ridx.at[b]], rows[b], gsem[b])

    def scatter(b):
        pltpu.async_copy(rows[b], acc.at[cidx.at[b]], ssem[b], add=True)

    def wait_idx(sem_ring, ring, b):
        pltpu.make_async_copy(row_hbm.at[base], ring.at[b],
                              sem_ring[b]).wait()

    def wait_gather(b):
        pltpu.make_async_copy(h_hbm.at[ridx.at[b]], rows[b], gsem[b]).wait()

    def wait_scatter(b):
        pltpu.make_async_copy(rows[b], acc.at[cidx.at[b]], ssem[b]).wait()

    zcp = pltpu.async_copy(zeros_hbm, acc.at[pl.ds(sid * RPT, RPT)], zsem)
    for b in range(NBUF):
        load_ridx(b, b)
    for b in range(2):
        load_cidx(b, b)
    for b in range(2):
        wait_idx(rsem, ridx, b)
        gather(b)
    zcp.wait()
    plsc.subcore_barrier()

    def outer(g, carry):
        for b in range(NBUF):
            i = g * NBUF + b
            bg = (b + 2) % NBUF
            wait_gather(b)
            wait_idx(csem, cidx, b)
            scatter(b)

            @pl.when(i + NBUF < CHUNKS)
            def _reload_ridx():
                load_ridx(i + NBUF, b)

            @pl.when(i >= 2)
            def _drain_scatter():
                wait_scatter(bg)

            @pl.when(i + 2 < CHUNKS)
            def _next_gather():
                load_cidx(i + 2, bg)
                wait_idx(rsem, ridx, bg)
                gather(bg)

        return carry

    lax.fori_loop(0, CHUNKS // NBUF, outer, 0)
    wait_scatter((CHUNKS - 2) % NBUF)
    wait_scatter((CHUNKS - 1) % NBUF)
    plsc.subcore_barrier()
    pltpu.sync_copy(acc.at[pl.ds(sid * RPT, RPT)],
                    out_hbm.at[cid, pl.ds(sid * RPT, RPT)])


def _make_prop_kernel(d):
    return pl.kernel(
        _prop_body,
        out_type=jax.ShapeDtypeStruct((NC, N_PAD, d), jnp.float32),
        mesh=_sc_mesh(),
        scratch_types=[
            pltpu.VMEM((NBUF, K), jnp.int32),
            pltpu.VMEM((NBUF, K), jnp.int32),
            pltpu.VMEM((K, d), jnp.float32),
            pltpu.VMEM((K, d), jnp.float32),
            pltpu.VMEM((K, d), jnp.float32),
            pltpu.VMEM((K, d), jnp.float32),
            pltpu.VMEM_SHARED((N_PAD, d), jnp.float32),
        ] + [pltpu.SemaphoreType.DMA] * 17,
    )


def _dis_body(p_ref, o_ref):
    deg = p_ref[0] + p_ref[1] + 1.0
    o_ref[...] = lax.rsqrt(deg)


def _mm1_body(x_ref, d_ref, w_ref, o_ref):
    xs = x_ref[...] * d_ref[...]
    o_ref[...] = jnp.dot(xs, w_ref[...], preferred_element_type=jnp.float32)


def _epi1_body(p0_ref, p1_ref, h_ref, d_ref, b_ref, w_ref, o_ref):
    s = p0_ref[...] + p1_ref[...] + h_ref[...]
    d = d_ref[...]
    h1 = jnp.maximum(d * s + b_ref[...], 0.0)
    o_ref[...] = jnp.dot(d * h1, w_ref[...],
                         preferred_element_type=jnp.float32)


def _epi2_body(p0_ref, p1_ref, h_ref, d_ref, b_ref, o_ref):
    s = p0_ref[:, :N_CLASSES] + p1_ref[:, :N_CLASSES] + h_ref[:, :N_CLASSES]
    z = d_ref[...] * s + b_ref[...]
    m = jnp.max(z, axis=1, keepdims=True)
    e = jnp.exp(z - m)
    lse = jnp.log(jnp.sum(e, axis=1, keepdims=True))
    o_ref[...] = z - m - lse


_R = RPT
_G = N_PAD // _R


def _row_specs(d):
    return pl.BlockSpec((_R, d), lambda i: (i, 0))


def _tc_mm1(x, dis1, w):
    return pl.pallas_call(
        _mm1_body,
        grid=(_G,),
        in_specs=[_row_specs(D_FEAT), pl.BlockSpec((_R, 1), lambda i: (i, 0)),
                  pl.BlockSpec((D_FEAT, D_HID), lambda i: (0, 0))],
        out_specs=_row_specs(D_HID),
        out_shape=jax.ShapeDtypeStruct((N_PAD, D_HID), jnp.float32),
    )(x, dis1, w)


def _tc_epi1(p0, p1, h, dis1, b, w):
    return pl.pallas_call(
        _epi1_body,
        grid=(_G,),
        in_specs=[_row_specs(D_HID), _row_specs(D_HID), _row_specs(D_HID),
                  pl.BlockSpec((_R, 1), lambda i: (i, 0)),
                  pl.BlockSpec((1, D_HID), lambda i: (0, 0)),
                  pl.BlockSpec((D_HID, D_HID), lambda i: (0, 0))],
        out_specs=_row_specs(D_HID),
        out_shape=jax.ShapeDtypeStruct((N_PAD, D_HID), jnp.float32),
    )(p0, p1, h, dis1, b, w)


def _tc_epi2(p0, p1, h, dis1, b):
    return pl.pallas_call(
        _epi2_body,
        grid=(_G,),
        in_specs=[_row_specs(D_HID), _row_specs(D_HID), _row_specs(D_HID),
                  pl.BlockSpec((_R, 1), lambda i: (i, 0)),
                  pl.BlockSpec((1, N_CLASSES), lambda i: (0, 0))],
        out_specs=_row_specs(N_CLASSES),
        out_shape=jax.ShapeDtypeStruct((N_PAD, N_CLASSES), jnp.float32),
    )(p0, p1, h, dis1, b)


def _tc_dis(parts):
    return pl.pallas_call(
        _dis_body,
        out_shape=jax.ShapeDtypeStruct((DEG_PAD // 128, 128), jnp.float32),
    )(parts)


def kernel(x, edge_index, W1, b1, W2, b2):
    assert x.shape == (N_NODES, D_FEAT)
    assert edge_index.shape == (2, N_EDGES)

    row2d = edge_index[0].reshape(NW * CHUNKS, K)
    col2d = edge_index[1].reshape(NW * CHUNKS, K)
    col2d_deg = edge_index[1].reshape(NW, DCH, DK)
    x_pad = jnp.pad(x, ((0, N_PAD - N_NODES), (0, 0)))
    zeros1 = jnp.zeros((DRPT,), jnp.float32)
    zeros128 = jnp.zeros((RPT, D_HID), jnp.float32)
    W2p = jnp.pad(W2, ((0, 0), (0, D_HID - N_CLASSES)))

    deg_parts = _make_deg_kernel()(col2d_deg, zeros1)
    dis_full = _tc_dis(deg_parts.reshape(NC, DEG_PAD // 128, 128))
    dis1 = dis_full.reshape(DEG_PAD, 1)[:N_PAD]

    h1p = _tc_mm1(x_pad, dis1, W1)
    parts1 = _make_prop_kernel(D_HID)(h1p, row2d, col2d, zeros128)
    h2p = _tc_epi1(parts1[0], parts1[1], h1p, dis1,
                   b1.reshape(1, D_HID), W2p)
    parts2 = _make_prop_kernel(D_HID)(h2p, row2d, col2d, zeros128)
    out = _tc_epi2(parts2[0], parts2[1], h2p, dis1,
                   b2.reshape(1, N_CLASSES))
    return out[:N_NODES]


# speedup vs baseline: 31.1171x; 1.1648x over previous
import jax
import jax.numpy as jnp
from jax import lax
from jax.experimental import pallas as pl
from jax.experimental.pallas import tpu as pltpu
from jax.experimental.pallas import tpu_sc as plsc

N_NODES = 10000
N_EDGES = 320000
D_FEAT = 128
D_HID = 128
N_CLASSES = 64

NC = 2
NS = 16
NW = NC * NS
EPW = N_EDGES // NW

N_PAD = 10112
RPT = N_PAD // NS

K = 50
CHUNKS = EPW // K
NBUF = 5
G = NBUF - 2

DEG_PAD = 10240
DRPT = DEG_PAD // NS
DK = 80
DCH = (N_EDGES // NS) // DK


def _sc_mesh():
    return plsc.VectorSubcoreMesh(core_axis_name="c", subcore_axis_name="s")


def _deg_body(col_hbm, zeros_hbm, out_hbm, cidx2, ones, dbuf, acc,
              isem, ssem):
    cid = lax.axis_index("c")
    sid = lax.axis_index("s")

    @pl.when(cid == 0)
    def _count():
        idx_cp = pltpu.async_copy(col_hbm.at[sid], cidx2, isem)
        pltpu.sync_copy(zeros_hbm, acc.at[pl.ds(sid * DRPT, DRPT)])
        for j in range(DK // 16):
            ones[pl.ds(j * 16, 16)] = jnp.ones((16,), jnp.float32)
        idx_cp.wait()
        plsc.subcore_barrier()

        LAG = 100

        def fire(i, carry):
            pltpu.async_copy(ones, acc.at[cidx2.at[i]], ssem, add=True)

            @pl.when(i >= LAG)
            def _():
                pltpu.make_async_copy(ones, acc.at[cidx2.at[i]],
                                      ssem).wait()

            return carry

        def drain(i, carry):
            pltpu.make_async_copy(ones, acc.at[cidx2.at[i]], ssem).wait()
            return carry

        lax.fori_loop(0, DCH, fire, 0)
        lax.fori_loop(0, LAG, drain, 0)
        plsc.subcore_barrier()

        pltpu.sync_copy(acc.at[pl.ds(sid * DRPT, DRPT)], dbuf)

        def rsqrt16(j, carry):
            d = dbuf[pl.ds(j * 16, 16)] + 1.0
            i0 = lax.bitcast_convert_type(d, jnp.int32)
            i1 = jnp.int32(0x5F3759DF) - lax.shift_right_logical(i0, 1)
            y = lax.bitcast_convert_type(i1, jnp.float32)
            for _ in range(3):
                y = y * (1.5 - 0.5 * d * y * y)
            dbuf[pl.ds(j * 16, 16)] = y
            return carry

        lax.fori_loop(0, DRPT // 16, rsqrt16, 0)
        pltpu.sync_copy(dbuf, out_hbm.at[pl.ds(sid * DRPT, DRPT)])


def _make_deg_kernel():
    return pl.kernel(
        _deg_body,
        out_type=jax.ShapeDtypeStruct((DEG_PAD,), jnp.float32),
        mesh=_sc_mesh(),
        scratch_types=[
            pltpu.VMEM((DCH, DK), jnp.int32),
            pltpu.VMEM((DK,), jnp.float32),
            pltpu.VMEM((DRPT,), jnp.float32),
            pltpu.VMEM_SHARED((DEG_PAD,), jnp.float32),
            pltpu.SemaphoreType.DMA,
            pltpu.SemaphoreType.DMA,
        ],
    )


def _prop_body(h_hbm, row_hbm, col_hbm, zeros_hbm, out_hbm,
               ridx, cidx, rows0, rows1, rows2, rows3, rows4, zsem,
               rsem0, rsem1, rsem2, rsem3, rsem4,
               csem0, csem1, csem2, csem3, csem4,
               gsem0, gsem1, gsem2, gsem3, gsem4,
               ssem0, ssem1, ssem2, ssem3, ssem4, acc):
    cid = lax.axis_index("c")
    sid = lax.axis_index("s")
    wid = cid * NS + sid
    base = wid * CHUNKS
    rows = (rows0, rows1, rows2, rows3, rows4)
    rsem = (rsem0, rsem1, rsem2, rsem3, rsem4)
    csem = (csem0, csem1, csem2, csem3, csem4)
    gsem = (gsem0, gsem1, gsem2, gsem3, gsem4)
    ssem = (ssem0, ssem1, ssem2, ssem3, ssem4)

    def load_ridx(i, b):
        pltpu.async_copy(row_hbm.at[base + i], ridx.at[b], rsem[b])

    def load_cidx(i, b):
        pltpu.async_copy(col_hbm.at[base + i], cidx.at[b], csem[b])

    def gather(b):
        pltpu.async_copy(h_hbm.at[ridx.at[b]], rows[b], gsem[b])

    def scatter(b):
        pltpu.async_copy(rows[b], acc.at[cidx.at[b]], ssem[b], add=True)

    def wait_idx(sem_ring, ring, b):
        pltpu.make_async_copy(row_hbm.at[base], ring.at[b],
                              sem_ring[b]).wait()

    def wait_gather(b):
        pltpu.make_async_copy(h_hbm.at[ridx.at[b]], rows[b], gsem[b]).wait()

    def wait_scatter(b):
        pltpu.make_async_copy(rows[b], acc.at[cidx.at[b]], ssem[b]).wait()

    zcp = pltpu.async_copy(zeros_hbm, acc.at[pl.ds(sid * RPT, RPT)], zsem)
    for b in range(NBUF):
        load_ridx(b, b)
    for b in range(G):
        load_cidx(b, b)
    for b in range(G):
        wait_idx(rsem, ridx, b)
        gather(b)
    zcp.wait()
    plsc.subcore_barrier()

    def outer(g, carry):
        for b in range(NBUF):
            i = g * NBUF + b
            bg = (b + G) % NBUF
            wait_gather(b)
            wait_idx(csem, cidx, b)
            scatter(b)

            @pl.when(i + NBUF < CHUNKS)
            def _reload_ridx():
                load_ridx(i + NBUF, b)

            @pl.when(i >= NBUF - G)
            def _drain_scatter():
                wait_scatter(bg)

            @pl.when(i + G < CHUNKS)
            def _next_gather():
                load_cidx(i + G, bg)
                wait_idx(rsem, ridx, bg)
                gather(bg)

        return carry

    lax.fori_loop(0, CHUNKS // NBUF, outer, 0)
    wait_scatter((CHUNKS - 2) % NBUF)
    wait_scatter((CHUNKS - 1) % NBUF)
    plsc.subcore_barrier()
    pltpu.sync_copy(acc.at[pl.ds(sid * RPT, RPT)],
                    out_hbm.at[cid, pl.ds(sid * RPT, RPT)])


def _make_prop_kernel(d):
    return pl.kernel(
        _prop_body,
        out_type=jax.ShapeDtypeStruct((NC, N_PAD, d), jnp.float32),
        mesh=_sc_mesh(),
        scratch_types=[
            pltpu.VMEM((NBUF, K), jnp.int32),
            pltpu.VMEM((NBUF, K), jnp.int32),
        ] + [pltpu.VMEM((K, d), jnp.float32)] * NBUF
          + [pltpu.SemaphoreType.DMA] * (1 + 4 * NBUF)
          + [pltpu.VMEM_SHARED((N_PAD, d), jnp.float32)],
    )


def _mm1_body(x_ref, d_ref, w_ref, o_ref):
    xs = x_ref[...] * d_ref[...]
    o_ref[...] = jnp.dot(xs, w_ref[...], preferred_element_type=jnp.float32)


def _epi1_body(p0_ref, p1_ref, h_ref, d_ref, b_ref, w_ref, o_ref):
    s = p0_ref[...] + p1_ref[...] + h_ref[...]
    d = d_ref[...]
    h1 = jnp.maximum(d * s + b_ref[...], 0.0)
    o_ref[...] = jnp.dot(d * h1, w_ref[...],
                         preferred_element_type=jnp.float32)


def _epi2_body(p0_ref, p1_ref, h_ref, d_ref, b_ref, o_ref):
    s = p0_ref[:, :N_CLASSES] + p1_ref[:, :N_CLASSES] + h_ref[:, :N_CLASSES]
    z = d_ref[...] * s + b_ref[...]
    m = jnp.max(z, axis=1, keepdims=True)
    e = jnp.exp(z - m)
    lse = jnp.log(jnp.sum(e, axis=1, keepdims=True))
    o_ref[...] = z - m - lse


_R = RPT
_G = N_PAD // _R


def _row_specs(d):
    return pl.BlockSpec((_R, d), lambda i: (i, 0))


def _tc_mm1(x, dis1, w):
    return pl.pallas_call(
        _mm1_body,
        grid=(_G,),
        in_specs=[_row_specs(D_FEAT), pl.BlockSpec((_R, 1), lambda i: (i, 0)),
                  pl.BlockSpec((D_FEAT, D_HID), lambda i: (0, 0))],
        out_specs=_row_specs(D_HID),
        out_shape=jax.ShapeDtypeStruct((N_PAD, D_HID), jnp.float32),
    )(x, dis1, w)


def _tc_epi1(p0, p1, h, dis1, b, w):
    return pl.pallas_call(
        _epi1_body,
        grid=(_G,),
        in_specs=[_row_specs(D_HID), _row_specs(D_HID), _row_specs(D_HID),
                  pl.BlockSpec((_R, 1), lambda i: (i, 0)),
                  pl.BlockSpec((1, D_HID), lambda i: (0, 0)),
                  pl.BlockSpec((D_HID, D_HID), lambda i: (0, 0))],
        out_specs=_row_specs(D_HID),
        out_shape=jax.ShapeDtypeStruct((N_PAD, D_HID), jnp.float32),
    )(p0, p1, h, dis1, b, w)


def _tc_epi2(p0, p1, h, dis1, b):
    return pl.pallas_call(
        _epi2_body,
        grid=(_G,),
        in_specs=[_row_specs(D_HID), _row_specs(D_HID), _row_specs(D_HID),
                  pl.BlockSpec((_R, 1), lambda i: (i, 0)),
                  pl.BlockSpec((1, N_CLASSES), lambda i: (0, 0))],
        out_specs=_row_specs(N_CLASSES),
        out_shape=jax.ShapeDtypeStruct((N_PAD, N_CLASSES), jnp.float32),
    )(p0, p1, h, dis1, b)


def kernel(x, edge_index, W1, b1, W2, b2):
    assert x.shape == (N_NODES, D_FEAT)
    assert edge_index.shape == (2, N_EDGES)

    row2d = edge_index[0].reshape(NW * CHUNKS, K)
    col2d = edge_index[1].reshape(NW * CHUNKS, K)
    col2d_deg = edge_index[1].reshape(NS, DCH, DK)
    x_pad = jnp.pad(x, ((0, N_PAD - N_NODES), (0, 0)))
    zeros1 = jnp.zeros((DRPT,), jnp.float32)
    zeros128 = jnp.zeros((RPT, D_HID), jnp.float32)
    W2p = jnp.pad(W2, ((0, 0), (0, D_HID - N_CLASSES)))

    dis_full = _make_deg_kernel()(col2d_deg, zeros1)
    dis1 = dis_full.reshape(DEG_PAD, 1)[:N_PAD]

    h1p = _tc_mm1(x_pad, dis1, W1)
    parts1 = _make_prop_kernel(D_HID)(h1p, row2d, col2d, zeros128)
    h2p = _tc_epi1(parts1[0], parts1[1], h1p, dis1,
                   b1.reshape(1, D_HID), W2p)
    parts2 = _make_prop_kernel(D_HID)(h2p, row2d, col2d, zeros128)
    out = _tc_epi2(parts2[0], parts2[1], h2p, dis1,
                   b2.reshape(1, N_CLASSES))
    return out[:N_NODES]


# speedup vs baseline: 32.9802x; 1.0599x over previous
import jax
import jax.numpy as jnp
from jax import lax
from jax.experimental import pallas as pl
from jax.experimental.pallas import tpu as pltpu
from jax.experimental.pallas import tpu_sc as plsc

N_NODES = 10000
N_EDGES = 320000
D_FEAT = 128
D_HID = 128
N_CLASSES = 64

NC = 2
NS = 16
NW = NC * NS
EPW = N_EDGES // NW

N_PAD = 10112
RPT = N_PAD // NS

K = 50
CHUNKS = EPW // K
NBUF = 5
G = NBUF - 2

DEG_PAD = 10240
DRPT = DEG_PAD // NS
DK = 80
DCH = (N_EDGES // NS) // DK


def _sc_mesh():
    return plsc.VectorSubcoreMesh(core_axis_name="c", subcore_axis_name="s")


def _deg_body(col_hbm, zeros_hbm, out_hbm, cidx2, ones, dbuf, acc,
              isem, ssem):
    cid = lax.axis_index("c")
    sid = lax.axis_index("s")

    @pl.when(cid == 0)
    def _count():
        idx_cp = pltpu.async_copy(col_hbm.at[sid], cidx2, isem)
        pltpu.sync_copy(zeros_hbm, acc.at[pl.ds(sid * DRPT, DRPT)])
        for j in range(DK // 16):
            ones[pl.ds(j * 16, 16)] = jnp.ones((16,), jnp.float32)
        idx_cp.wait()
        plsc.subcore_barrier()

        LAG = 100

        def fire(i, carry):
            pltpu.async_copy(ones, acc.at[cidx2.at[i]], ssem, add=True)

            @pl.when(i >= LAG)
            def _():
                pltpu.make_async_copy(ones, acc.at[cidx2.at[i]],
                                      ssem).wait()

            return carry

        def drain(i, carry):
            pltpu.make_async_copy(ones, acc.at[cidx2.at[i]], ssem).wait()
            return carry

        lax.fori_loop(0, DCH, fire, 0)
        lax.fori_loop(0, LAG, drain, 0)
        plsc.subcore_barrier()

        pltpu.sync_copy(acc.at[pl.ds(sid * DRPT, DRPT)], dbuf)

        def rsqrt16(j, carry):
            d = dbuf[pl.ds(j * 16, 16)] + 1.0
            i0 = lax.bitcast_convert_type(d, jnp.int32)
            i1 = jnp.int32(0x5F3759DF) - lax.shift_right_logical(i0, 1)
            y = lax.bitcast_convert_type(i1, jnp.float32)
            for _ in range(3):
                y = y * (1.5 - 0.5 * d * y * y)
            dbuf[pl.ds(j * 16, 16)] = y
            return carry

        lax.fori_loop(0, DRPT // 16, rsqrt16, 0)
        pltpu.sync_copy(dbuf, out_hbm.at[pl.ds(sid * DRPT, DRPT)])


def _make_deg_kernel():
    return pl.kernel(
        _deg_body,
        out_type=jax.ShapeDtypeStruct((DEG_PAD,), jnp.float32),
        mesh=_sc_mesh(),
        scratch_types=[
            pltpu.VMEM((DCH, DK), jnp.int32),
            pltpu.VMEM((DK,), jnp.float32),
            pltpu.VMEM((DRPT,), jnp.float32),
            pltpu.VMEM_SHARED((DEG_PAD,), jnp.float32),
            pltpu.SemaphoreType.DMA,
            pltpu.SemaphoreType.DMA,
        ],
    )


def _prop_body(h_hbm, row_hbm, col_hbm, zeros_hbm, out_hbm,
               ridx, cidx, rows0, rows1, rows2, rows3, rows4, zsem,
               rsem0, rsem1, rsem2, rsem3, rsem4,
               csem0, csem1, csem2, csem3, csem4,
               gsem0, gsem1, gsem2, gsem3, gsem4,
               ssem0, ssem1, ssem2, ssem3, ssem4, acc):
    cid = lax.axis_index("c")
    sid = lax.axis_index("s")
    wid = cid * NS + sid
    base = wid * CHUNKS
    rows = (rows0, rows1, rows2, rows3, rows4)
    rsem = (rsem0, rsem1, rsem2, rsem3, rsem4)
    csem = (csem0, csem1, csem2, csem3, csem4)
    gsem = (gsem0, gsem1, gsem2, gsem3, gsem4)
    ssem = (ssem0, ssem1, ssem2, ssem3, ssem4)

    def load_ridx(i, b):
        pltpu.async_copy(row_hbm.at[base + i], ridx.at[b], rsem[b])

    def load_cidx(i, b):
        pltpu.async_copy(col_hbm.at[base + i], cidx.at[b], csem[b])

    def gather(b):
        pltpu.async_copy(h_hbm.at[ridx.at[b]], rows[b], gsem[b])

    def scatter(b):
        pltpu.async_copy(rows[b], acc.at[cidx.at[b]], ssem[b], add=True)

    def wait_idx(sem_ring, ring, b):
        pltpu.make_async_copy(row_hbm.at[base], ring.at[b],
                              sem_ring[b]).wait()

    def wait_gather(b):
        pltpu.make_async_copy(h_hbm.at[ridx.at[b]], rows[b], gsem[b]).wait()

    def wait_scatter(b):
        pltpu.make_async_copy(rows[b], acc.at[cidx.at[b]], ssem[b]).wait()

    zcp = pltpu.async_copy(zeros_hbm, acc.at[pl.ds(sid * RPT, RPT)], zsem)
    for b in range(NBUF):
        load_ridx(b, b)
    for b in range(G):
        load_cidx(b, b)
    for b in range(G):
        wait_idx(rsem, ridx, b)
        gather(b)
    zcp.wait()
    plsc.subcore_barrier()

    def outer(g, carry):
        for b in range(NBUF):
            i = g * NBUF + b
            bg = (b + G) % NBUF
            wait_gather(b)
            wait_idx(csem, cidx, b)
            scatter(b)

            @pl.when(i + NBUF < CHUNKS)
            def _reload_ridx():
                load_ridx(i + NBUF, b)

            @pl.when(i >= NBUF - G)
            def _drain_scatter():
                wait_scatter(bg)

            @pl.when(i + G < CHUNKS)
            def _next_gather():
                load_cidx(i + G, bg)
                wait_idx(rsem, ridx, bg)
                gather(bg)

        return carry

    lax.fori_loop(0, CHUNKS // NBUF, outer, 0)
    wait_scatter((CHUNKS - 2) % NBUF)
    wait_scatter((CHUNKS - 1) % NBUF)
    plsc.subcore_barrier()
    pltpu.sync_copy(acc.at[pl.ds(sid * RPT, RPT)],
                    out_hbm.at[cid, pl.ds(sid * RPT, RPT)])


def _make_prop_kernel(d):
    return pl.kernel(
        _prop_body,
        out_type=jax.ShapeDtypeStruct((NC, N_PAD, d), jnp.float32),
        mesh=_sc_mesh(),
        compiler_params=pltpu.CompilerParams(use_tc_tiling_on_sc=False)
        if d != 128 else None,
        scratch_types=[
            pltpu.VMEM((NBUF, K), jnp.int32),
            pltpu.VMEM((NBUF, K), jnp.int32),
        ] + [pltpu.VMEM((K, d), jnp.float32)] * NBUF
          + [pltpu.SemaphoreType.DMA] * (1 + 4 * NBUF)
          + [pltpu.VMEM_SHARED((N_PAD, d), jnp.float32)],
    )


def _mm1_body(x_ref, d_ref, w_ref, o_ref):
    xs = x_ref[...] * d_ref[...]
    o_ref[...] = jnp.dot(xs, w_ref[...], preferred_element_type=jnp.float32)


def _epi1_body(p0_ref, p1_ref, h_ref, d_ref, b_ref, w_ref, o_ref):
    s = p0_ref[...] + p1_ref[...] + h_ref[...]
    d = d_ref[...]
    h1 = jnp.maximum(d * s + b_ref[...], 0.0)
    o_ref[...] = jnp.dot(d * h1, w_ref[...],
                         preferred_element_type=jnp.float32)


def _epi2_body(p0_ref, p1_ref, h_ref, d_ref, b_ref, o_ref):
    s = p0_ref[...] + p1_ref[...] + h_ref[...]
    z = d_ref[...] * s + b_ref[...]
    m = jnp.max(z, axis=1, keepdims=True)
    e = jnp.exp(z - m)
    lse = jnp.log(jnp.sum(e, axis=1, keepdims=True))
    o_ref[...] = z - m - lse


_R = RPT
_G = N_PAD // _R


def _row_specs(d):
    return pl.BlockSpec((_R, d), lambda i: (i, 0))


def _tc_mm1(x, dis1, w):
    return pl.pallas_call(
        _mm1_body,
        grid=(_G,),
        in_specs=[_row_specs(D_FEAT), pl.BlockSpec((_R, 1), lambda i: (i, 0)),
                  pl.BlockSpec((D_FEAT, D_HID), lambda i: (0, 0))],
        out_specs=_row_specs(D_HID),
        out_shape=jax.ShapeDtypeStruct((N_PAD, D_HID), jnp.float32),
    )(x, dis1, w)


def _tc_epi1(p0, p1, h, dis1, b, w):
    return pl.pallas_call(
        _epi1_body,
        grid=(_G,),
        in_specs=[_row_specs(D_HID), _row_specs(D_HID), _row_specs(D_HID),
                  pl.BlockSpec((_R, 1), lambda i: (i, 0)),
                  pl.BlockSpec((1, D_HID), lambda i: (0, 0)),
                  pl.BlockSpec((D_HID, N_CLASSES), lambda i: (0, 0))],
        out_specs=_row_specs(N_CLASSES),
        out_shape=jax.ShapeDtypeStruct((N_PAD, N_CLASSES), jnp.float32),
    )(p0, p1, h, dis1, b, w)


def _tc_epi2(p0, p1, h, dis1, b):
    return pl.pallas_call(
        _epi2_body,
        grid=(_G,),
        in_specs=[_row_specs(N_CLASSES), _row_specs(N_CLASSES),
                  _row_specs(N_CLASSES),
                  pl.BlockSpec((_R, 1), lambda i: (i, 0)),
                  pl.BlockSpec((1, N_CLASSES), lambda i: (0, 0))],
        out_specs=_row_specs(N_CLASSES),
        out_shape=jax.ShapeDtypeStruct((N_PAD, N_CLASSES), jnp.float32),
    )(p0, p1, h, dis1, b)


def kernel(x, edge_index, W1, b1, W2, b2):
    assert x.shape == (N_NODES, D_FEAT)
    assert edge_index.shape == (2, N_EDGES)

    row2d = edge_index[0].reshape(NW * CHUNKS, K)
    col2d = edge_index[1].reshape(NW * CHUNKS, K)
    col2d_deg = edge_index[1].reshape(NS, DCH, DK)
    x_pad = jnp.pad(x, ((0, N_PAD - N_NODES), (0, 0)))
    zeros1 = jnp.zeros((DRPT,), jnp.float32)
    zeros128 = jnp.zeros((RPT, D_HID), jnp.float32)
    zeros64 = jnp.zeros((RPT, N_CLASSES), jnp.float32)

    dis_full = _make_deg_kernel()(col2d_deg, zeros1)
    dis1 = dis_full.reshape(DEG_PAD, 1)[:N_PAD]

    h1p = _tc_mm1(x_pad, dis1, W1)
    parts1 = _make_prop_kernel(D_HID)(h1p, row2d, col2d, zeros128)
    h2p = _tc_epi1(parts1[0], parts1[1], h1p, dis1,
                   b1.reshape(1, D_HID), W2)
    parts2 = _make_prop_kernel(N_CLASSES)(h2p, row2d, col2d, zeros64)
    out = _tc_epi2(parts2[0], parts2[1], h2p, dis1,
                   b2.reshape(1, N_CLASSES))
    return out[:N_NODES]


# speedup vs baseline: 33.9724x; 1.0301x over previous
import jax
import jax.numpy as jnp
from jax import lax
from jax.experimental import pallas as pl
from jax.experimental.pallas import tpu as pltpu
from jax.experimental.pallas import tpu_sc as plsc

N_NODES = 10000
N_EDGES = 320000
D_FEAT = 128
D_HID = 128
N_CLASSES = 64

NC = 2
NS = 16
NW = NC * NS
EPW = N_EDGES // NW

N_PAD = 10112
RPT = N_PAD // NS

K = 40
CHUNKS = EPW // K
NBUF = 5
G = NBUF - 2

DEG_PAD = 10240
DRPT = DEG_PAD // NS
DK = 80
DCH = (N_EDGES // NS) // DK


def _sc_mesh():
    return plsc.VectorSubcoreMesh(core_axis_name="c", subcore_axis_name="s")


def _deg_body(col_hbm, zeros_hbm, out_hbm, cidx2, ones, dbuf, acc,
              isem, ssem):
    cid = lax.axis_index("c")
    sid = lax.axis_index("s")

    @pl.when(cid == 0)
    def _count():
        idx_cp = pltpu.async_copy(col_hbm.at[sid], cidx2, isem)
        pltpu.sync_copy(zeros_hbm, acc.at[pl.ds(sid * DRPT, DRPT)])
        for j in range(DK // 16):
            ones[pl.ds(j * 16, 16)] = jnp.ones((16,), jnp.float32)
        idx_cp.wait()
        plsc.subcore_barrier()

        LAG = 100

        def fire(i, carry):
            pltpu.async_copy(ones, acc.at[cidx2.at[i]], ssem, add=True)

            @pl.when(i >= LAG)
            def _():
                pltpu.make_async_copy(ones, acc.at[cidx2.at[i]],
                                      ssem).wait()

            return carry

        def drain(i, carry):
            pltpu.make_async_copy(ones, acc.at[cidx2.at[i]], ssem).wait()
            return carry

        lax.fori_loop(0, DCH, fire, 0)
        lax.fori_loop(0, LAG, drain, 0)
        plsc.subcore_barrier()

        pltpu.sync_copy(acc.at[pl.ds(sid * DRPT, DRPT)], dbuf)

        def rsqrt16(j, carry):
            d = dbuf[pl.ds(j * 16, 16)] + 1.0
            i0 = lax.bitcast_convert_type(d, jnp.int32)
            i1 = jnp.int32(0x5F3759DF) - lax.shift_right_logical(i0, 1)
            y = lax.bitcast_convert_type(i1, jnp.float32)
            for _ in range(3):
                y = y * (1.5 - 0.5 * d * y * y)
            dbuf[pl.ds(j * 16, 16)] = y
            return carry

        lax.fori_loop(0, DRPT // 16, rsqrt16, 0)
        pltpu.sync_copy(dbuf, out_hbm.at[pl.ds(sid * DRPT, DRPT)])


def _make_deg_kernel():
    return pl.kernel(
        _deg_body,
        out_type=jax.ShapeDtypeStruct((DEG_PAD,), jnp.float32),
        mesh=_sc_mesh(),
        scratch_types=[
            pltpu.VMEM((DCH, DK), jnp.int32),
            pltpu.VMEM((DK,), jnp.float32),
            pltpu.VMEM((DRPT,), jnp.float32),
            pltpu.VMEM_SHARED((DEG_PAD,), jnp.float32),
            pltpu.SemaphoreType.DMA,
            pltpu.SemaphoreType.DMA,
        ],
    )


def _prop_body(h_hbm, row_hbm, col_hbm, zeros_hbm, out_hbm,
               ridx, cidx, rows0, rows1, rows2, rows3, rows4, zsem,
               rsem0, rsem1, rsem2, rsem3, rsem4,
               csem0, csem1, csem2, csem3, csem4,
               gsem0, gsem1, gsem2, gsem3, gsem4,
               ssem0, ssem1, ssem2, ssem3, ssem4, acc):
    cid = lax.axis_index("c")
    sid = lax.axis_index("s")
    wid = cid * NS + sid
    base = wid * CHUNKS
    rows = (rows0, rows1, rows2, rows3, rows4)
    rsem = (rsem0, rsem1, rsem2, rsem3, rsem4)
    csem = (csem0, csem1, csem2, csem3, csem4)
    gsem = (gsem0, gsem1, gsem2, gsem3, gsem4)
    ssem = (ssem0, ssem1, ssem2, ssem3, ssem4)

    def load_ridx(i, b):
        pltpu.async_copy(row_hbm.at[pl.ds((base + i) * K, K)], ridx.at[b],
                         rsem[b])

    def load_cidx(i, b):
        pltpu.async_copy(col_hbm.at[pl.ds((base + i) * K, K)], cidx.at[b],
                         csem[b])

    def gather(b):
        pltpu.async_copy(h_hbm.at[ridx.at[b]], rows[b], gsem[b])

    def scatter(b):
        pltpu.async_copy(rows[b], acc.at[cidx.at[b]], ssem[b], add=True)

    def wait_idx(sem_ring, ring, b):
        pltpu.make_async_copy(row_hbm.at[pl.ds(0, K)], ring.at[b],
                              sem_ring[b]).wait()

    def wait_gather(b):
        pltpu.make_async_copy(h_hbm.at[ridx.at[b]], rows[b], gsem[b]).wait()

    def wait_scatter(b):
        pltpu.make_async_copy(rows[b], acc.at[cidx.at[b]], ssem[b]).wait()

    zcp = pltpu.async_copy(zeros_hbm, acc.at[pl.ds(sid * RPT, RPT)], zsem)
    for b in range(NBUF):
        load_ridx(b, b)
    for b in range(G):
        load_cidx(b, b)
    for b in range(G):
        wait_idx(rsem, ridx, b)
        gather(b)
    zcp.wait()
    plsc.subcore_barrier()

    def outer(g, carry):
        for b in range(NBUF):
            i = g * NBUF + b
            bg = (b + G) % NBUF
            wait_gather(b)
            wait_idx(csem, cidx, b)
            scatter(b)

            @pl.when(i + NBUF < CHUNKS)
            def _reload_ridx():
                load_ridx(i + NBUF, b)

            @pl.when(i >= NBUF - G)
            def _drain_scatter():
                wait_scatter(bg)

            @pl.when(i + G < CHUNKS)
            def _next_gather():
                load_cidx(i + G, bg)
                wait_idx(rsem, ridx, bg)
                gather(bg)

        return carry

    lax.fori_loop(0, CHUNKS // NBUF, outer, 0)
    wait_scatter((CHUNKS - 2) % NBUF)
    wait_scatter((CHUNKS - 1) % NBUF)
    plsc.subcore_barrier()
    pltpu.sync_copy(acc.at[pl.ds(sid * RPT, RPT)],
                    out_hbm.at[cid, pl.ds(sid * RPT, RPT)])


def _make_prop_kernel(d):
    return pl.kernel(
        _prop_body,
        out_type=jax.ShapeDtypeStruct((NC, N_PAD, d), jnp.float32),
        mesh=_sc_mesh(),
        compiler_params=pltpu.CompilerParams(use_tc_tiling_on_sc=False)
        if d != 128 else None,
        scratch_types=[
            pltpu.VMEM((NBUF, K), jnp.int32),
            pltpu.VMEM((NBUF, K), jnp.int32),
        ] + [pltpu.VMEM((K, d), jnp.float32)] * NBUF
          + [pltpu.SemaphoreType.DMA] * (1 + 4 * NBUF)
          + [pltpu.VMEM_SHARED((N_PAD, d), jnp.float32)],
    )


def _mm1_body(x_ref, d_ref, w_ref, o_ref):
    xs = x_ref[...] * d_ref[...]
    o_ref[...] = jnp.dot(xs, w_ref[...], preferred_element_type=jnp.float32)


def _epi1_body(p_ref, h_ref, d_ref, b_ref, w_ref, o_ref):
    s = p_ref[0] + p_ref[1] + h_ref[...]
    d = d_ref[...]
    h1 = jnp.maximum(d * s + b_ref[...], 0.0)
    o_ref[...] = jnp.dot(d * h1, w_ref[...],
                         preferred_element_type=jnp.float32)


def _epi2_body(p_ref, h_ref, d_ref, b_ref, o_ref):
    s = p_ref[0] + p_ref[1] + h_ref[...]
    z = d_ref[...] * s + b_ref[...]
    m = jnp.max(z, axis=1, keepdims=True)
    e = jnp.exp(z - m)
    lse = jnp.log(jnp.sum(e, axis=1, keepdims=True))
    o_ref[...] = z - m - lse


_R = RPT
_G = N_PAD // _R


def _row_specs(d):
    return pl.BlockSpec((_R, d), lambda i: (i, 0))


def _tc_mm1(x, dis1, w):
    return pl.pallas_call(
        _mm1_body,
        grid=(_G,),
        in_specs=[_row_specs(D_FEAT), pl.BlockSpec((_R, 1), lambda i: (i, 0)),
                  pl.BlockSpec((D_FEAT, D_HID), lambda i: (0, 0))],
        out_specs=_row_specs(D_HID),
        out_shape=jax.ShapeDtypeStruct((N_PAD, D_HID), jnp.float32),
    )(x, dis1, w)


def _tc_epi1(parts, h, dis1, b, w):
    return pl.pallas_call(
        _epi1_body,
        grid=(_G,),
        in_specs=[pl.BlockSpec((NC, _R, D_HID), lambda i: (0, i, 0)),
                  _row_specs(D_HID),
                  pl.BlockSpec((_R, 1), lambda i: (i, 0)),
                  pl.BlockSpec((1, D_HID), lambda i: (0, 0)),
                  pl.BlockSpec((D_HID, N_CLASSES), lambda i: (0, 0))],
        out_specs=_row_specs(N_CLASSES),
        out_shape=jax.ShapeDtypeStruct((N_PAD, N_CLASSES), jnp.float32),
    )(parts, h, dis1, b, w)


def _tc_epi2(parts, h, dis1, b):
    return pl.pallas_call(
        _epi2_body,
        grid=(_G,),
        in_specs=[pl.BlockSpec((NC, _R, N_CLASSES), lambda i: (0, i, 0)),
                  _row_specs(N_CLASSES),
                  pl.BlockSpec((_R, 1), lambda i: (i, 0)),
                  pl.BlockSpec((1, N_CLASSES), lambda i: (0, 0))],
        out_specs=_row_specs(N_CLASSES),
        out_shape=jax.ShapeDtypeStruct((N_PAD, N_CLASSES), jnp.float32),
    )(parts, h, dis1, b)


def kernel(x, edge_index, W1, b1, W2, b2):
    assert x.shape == (N_NODES, D_FEAT)
    assert edge_index.shape == (2, N_EDGES)

    row1d = edge_index[0]
    col1d = edge_index[1]
    col2d_deg = edge_index[1].reshape(NS, DCH, DK)
    x_pad = jnp.pad(x, ((0, N_PAD - N_NODES), (0, 0)))
    zeros1 = jnp.zeros((DRPT,), jnp.float32)
    zeros128 = jnp.zeros((RPT, D_HID), jnp.float32)
    zeros64 = jnp.zeros((RPT, N_CLASSES), jnp.float32)

    dis_full = _make_deg_kernel()(col2d_deg, zeros1)
    dis1 = dis_full.reshape(DEG_PAD, 1)[:N_PAD]

    h1p = _tc_mm1(x_pad, dis1, W1)
    parts1 = _make_prop_kernel(D_HID)(h1p, row1d, col1d, zeros128)
    h2p = _tc_epi1(parts1, h1p, dis1, b1.reshape(1, D_HID), W2)
    parts2 = _make_prop_kernel(N_CLASSES)(h2p, row1d, col1d, zeros64)
    out = _tc_epi2(parts2, h2p, dis1, b2.reshape(1, N_CLASSES))
    return out[:N_NODES]


# speedup vs baseline: 36.2141x; 1.0660x over previous
import jax
import jax.numpy as jnp
from jax import lax
from jax.experimental import pallas as pl
from jax.experimental.pallas import tpu as pltpu
from jax.experimental.pallas import tpu_sc as plsc

N_NODES = 10000
N_EDGES = 320000
D_FEAT = 128
D_HID = 128
N_CLASSES = 64

NC = 2
NS = 16
NW = NC * NS
EPW = N_EDGES // NW

N_PAD = 10112
RPT = N_PAD // NS

K = 40
CHUNKS = EPW // K
NBUF = 5
G = NBUF - 2

DEG_PAD = 10240
DRPT = DEG_PAD // NS
DK = 80
DCH = (N_EDGES // NS) // DK


def _sc_mesh():
    return plsc.VectorSubcoreMesh(core_axis_name="c", subcore_axis_name="s")


def _deg_body(col_hbm, zeros_hbm, out_hbm, cidx2, ones, dbuf, acc,
              isem, ssem):
    cid = lax.axis_index("c")
    sid = lax.axis_index("s")

    @pl.when(cid == 0)
    def _count():
        idx_cp = pltpu.async_copy(col_hbm.at[sid], cidx2, isem)
        pltpu.sync_copy(zeros_hbm, acc.at[pl.ds(sid * DRPT, DRPT)])
        for j in range(DK // 16):
            ones[pl.ds(j * 16, 16)] = jnp.ones((16,), jnp.float32)
        idx_cp.wait()
        plsc.subcore_barrier()

        LAG = 100

        def fire(i, carry):
            pltpu.async_copy(ones, acc.at[cidx2.at[i]], ssem, add=True)

            @pl.when(i >= LAG)
            def _():
                pltpu.make_async_copy(ones, acc.at[cidx2.at[i]],
                                      ssem).wait()

            return carry

        def drain(i, carry):
            pltpu.make_async_copy(ones, acc.at[cidx2.at[i]], ssem).wait()
            return carry

        lax.fori_loop(0, DCH, fire, 0)
        lax.fori_loop(0, LAG, drain, 0)
        plsc.subcore_barrier()

        pltpu.sync_copy(acc.at[pl.ds(sid * DRPT, DRPT)], dbuf)

        def rsqrt16(j, carry):
            d = dbuf[pl.ds(j * 16, 16)] + 1.0
            i0 = lax.bitcast_convert_type(d, jnp.int32)
            i1 = jnp.int32(0x5F3759DF) - lax.shift_right_logical(i0, 1)
            y = lax.bitcast_convert_type(i1, jnp.float32)
            for _ in range(3):
                y = y * (1.5 - 0.5 * d * y * y)
            dbuf[pl.ds(j * 16, 16)] = y
            return carry

        lax.fori_loop(0, DRPT // 16, rsqrt16, 0)
        pltpu.sync_copy(dbuf, out_hbm.at[pl.ds(sid * DRPT, DRPT)])


def _make_deg_kernel():
    return pl.kernel(
        _deg_body,
        out_type=jax.ShapeDtypeStruct((DEG_PAD,), jnp.float32),
        mesh=_sc_mesh(),
        scratch_types=[
            pltpu.VMEM((DCH, DK), jnp.int32),
            pltpu.VMEM((DK,), jnp.float32),
            pltpu.VMEM((DRPT,), jnp.float32),
            pltpu.VMEM_SHARED((DEG_PAD,), jnp.float32),
            pltpu.SemaphoreType.DMA,
            pltpu.SemaphoreType.DMA,
        ],
    )


def _make_prop_body(K, CHUNKS):
  def _prop_body(h_hbm, row_hbm, col_hbm, zeros_hbm, out_hbm,
                 ridx, cidx, rows0, rows1, rows2, rows3, rows4, zsem,
                 rsem0, rsem1, rsem2, rsem3, rsem4,
                 csem0, csem1, csem2, csem3, csem4,
                 gsem0, gsem1, gsem2, gsem3, gsem4,
                 ssem0, ssem1, ssem2, ssem3, ssem4, acc):
      cid = lax.axis_index("c")
      sid = lax.axis_index("s")
      wid = cid * NS + sid
      base = wid * CHUNKS
      rows = (rows0, rows1, rows2, rows3, rows4)
      rsem = (rsem0, rsem1, rsem2, rsem3, rsem4)
      csem = (csem0, csem1, csem2, csem3, csem4)
      gsem = (gsem0, gsem1, gsem2, gsem3, gsem4)
      ssem = (ssem0, ssem1, ssem2, ssem3, ssem4)

      def load_ridx(i, b):
          pltpu.async_copy(row_hbm.at[pl.ds((base + i) * K, K)], ridx.at[b],
                           rsem[b])

      def load_cidx(i, b):
          pltpu.async_copy(col_hbm.at[pl.ds((base + i) * K, K)], cidx.at[b],
                           csem[b])

      def gather(b):
          pltpu.async_copy(h_hbm.at[ridx.at[b]], rows[b], gsem[b])

      def scatter(b):
          pltpu.async_copy(rows[b], acc.at[cidx.at[b]], ssem[b], add=True)

      def wait_idx(sem_ring, ring, b):
          pltpu.make_async_copy(row_hbm.at[pl.ds(0, K)], ring.at[b],
                                sem_ring[b]).wait()

      def wait_gather(b):
          pltpu.make_async_copy(h_hbm.at[ridx.at[b]], rows[b], gsem[b]).wait()

      def wait_scatter(b):
          pltpu.make_async_copy(rows[b], acc.at[cidx.at[b]], ssem[b]).wait()

      zcp = pltpu.async_copy(zeros_hbm, acc.at[pl.ds(sid * RPT, RPT)], zsem)
      for b in range(NBUF):
          load_ridx(b, b)
      for b in range(G):
          load_cidx(b, b)
      for b in range(G):
          wait_idx(rsem, ridx, b)
          gather(b)
      zcp.wait()
      plsc.subcore_barrier()

      def outer(g, carry):
          for b in range(NBUF):
              i = g * NBUF + b
              bg = (b + G) % NBUF
              wait_gather(b)
              wait_idx(csem, cidx, b)
              scatter(b)

              @pl.when(i + NBUF < CHUNKS)
              def _reload_ridx():
                  load_ridx(i + NBUF, b)

              @pl.when(i >= NBUF - G)
              def _drain_scatter():
                  wait_scatter(bg)

              @pl.when(i + G < CHUNKS)
              def _next_gather():
                  load_cidx(i + G, bg)
                  wait_idx(rsem, ridx, bg)
                  gather(bg)

          return carry

      lax.fori_loop(0, CHUNKS // NBUF, outer, 0)
      wait_scatter((CHUNKS - 2) % NBUF)
      wait_scatter((CHUNKS - 1) % NBUF)
      plsc.subcore_barrier()
      pltpu.sync_copy(acc.at[pl.ds(sid * RPT, RPT)],
                      out_hbm.at[cid, pl.ds(sid * RPT, RPT)])
  return _prop_body


def _make_prop_kernel(d, k):
    chunks = EPW // k
    assert chunks % NBUF == 0
    return pl.kernel(
        _make_prop_body(k, chunks),
        out_type=jax.ShapeDtypeStruct((NC, N_PAD, d), jnp.float32),
        mesh=_sc_mesh(),
        compiler_params=pltpu.CompilerParams(use_tc_tiling_on_sc=False)
        if d != 128 else None,
        scratch_types=[
            pltpu.VMEM((NBUF, k), jnp.int32),
            pltpu.VMEM((NBUF, k), jnp.int32),
        ] + [pltpu.VMEM((k, d), jnp.float32)] * NBUF
          + [pltpu.SemaphoreType.DMA] * (1 + 4 * NBUF)
          + [pltpu.VMEM_SHARED((N_PAD, d), jnp.float32)],
    )


def _mm1_body(x_ref, d_ref, w_ref, o_ref):
    xs = x_ref[...] * d_ref[...]
    o_ref[...] = jnp.dot(xs, w_ref[...], preferred_element_type=jnp.float32)


def _epi1_body(p_ref, h_ref, d_ref, b_ref, w_ref, o_ref):
    s = p_ref[0] + p_ref[1] + h_ref[...]
    d = d_ref[...]
    h1 = jnp.maximum(d * s + b_ref[...], 0.0)
    o_ref[...] = jnp.dot(d * h1, w_ref[...],
                         preferred_element_type=jnp.float32)


def _epi2_body(p_ref, h_ref, d_ref, b_ref, o_ref):
    s = p_ref[0] + p_ref[1] + h_ref[...]
    z = d_ref[...] * s + b_ref[...]
    m = jnp.max(z, axis=1, keepdims=True)
    e = jnp.exp(z - m)
    lse = jnp.log(jnp.sum(e, axis=1, keepdims=True))
    o_ref[...] = z - m - lse


_R = RPT
_G = N_PAD // _R


def _row_specs(d):
    return pl.BlockSpec((_R, d), lambda i: (i, 0))


def _tc_mm1(x, dis1, w):
    return pl.pallas_call(
        _mm1_body,
        grid=(_G,),
        in_specs=[_row_specs(D_FEAT), pl.BlockSpec((_R, 1), lambda i: (i, 0)),
                  pl.BlockSpec((D_FEAT, D_HID), lambda i: (0, 0))],
        out_specs=_row_specs(D_HID),
        out_shape=jax.ShapeDtypeStruct((N_PAD, D_HID), jnp.float32),
    )(x, dis1, w)


def _tc_epi1(parts, h, dis1, b, w):
    return pl.pallas_call(
        _epi1_body,
        grid=(_G,),
        in_specs=[pl.BlockSpec((NC, _R, D_HID), lambda i: (0, i, 0)),
                  _row_specs(D_HID),
                  pl.BlockSpec((_R, 1), lambda i: (i, 0)),
                  pl.BlockSpec((1, D_HID), lambda i: (0, 0)),
                  pl.BlockSpec((D_HID, N_CLASSES), lambda i: (0, 0))],
        out_specs=_row_specs(N_CLASSES),
        out_shape=jax.ShapeDtypeStruct((N_PAD, N_CLASSES), jnp.float32),
    )(parts, h, dis1, b, w)


def _tc_epi2(parts, h, dis1, b):
    return pl.pallas_call(
        _epi2_body,
        grid=(_G,),
        in_specs=[pl.BlockSpec((NC, _R, N_CLASSES), lambda i: (0, i, 0)),
                  _row_specs(N_CLASSES),
                  pl.BlockSpec((_R, 1), lambda i: (i, 0)),
                  pl.BlockSpec((1, N_CLASSES), lambda i: (0, 0))],
        out_specs=_row_specs(N_CLASSES),
        out_shape=jax.ShapeDtypeStruct((N_PAD, N_CLASSES), jnp.float32),
    )(parts, h, dis1, b)


def kernel(x, edge_index, W1, b1, W2, b2):
    assert x.shape == (N_NODES, D_FEAT)
    assert edge_index.shape == (2, N_EDGES)

    row1d = edge_index[0]
    col1d = edge_index[1]
    col2d_deg = edge_index[1].reshape(NS, DCH, DK)
    x_pad = jnp.pad(x, ((0, N_PAD - N_NODES), (0, 0)))
    zeros1 = jnp.zeros((DRPT,), jnp.float32)
    zeros128 = jnp.zeros((RPT, D_HID), jnp.float32)
    zeros64 = jnp.zeros((RPT, N_CLASSES), jnp.float32)

    dis_full = _make_deg_kernel()(col2d_deg, zeros1)
    dis1 = dis_full.reshape(DEG_PAD, 1)[:N_PAD]

    h1p = _tc_mm1(x_pad, dis1, W1)
    parts1 = _make_prop_kernel(D_HID, 40)(h1p, row1d, col1d, zeros128)
    h2p = _tc_epi1(parts1, h1p, dis1, b1.reshape(1, D_HID), W2)
    parts2 = _make_prop_kernel(N_CLASSES, 80)(h2p, row1d, col1d, zeros64)
    out = _tc_epi2(parts2, h2p, dis1, b2.reshape(1, N_CLASSES))
    return out[:N_NODES]
